# baseline (speedup 1.0000x reference)
import jax
import jax.numpy as jnp
from jax.experimental import pallas as pl
from jax.experimental.pallas import tpu as pltpu


def _round_up(n, m):
    return (n + m - 1) // m * m


def _mlp_kernel(x_ref, w1_ref, b1_ref, w2_ref, b2_ref, w3_ref, b3_ref,
                w4_ref, b4_ref, out_ref):
    bf = jnp.bfloat16
    h = jnp.dot(x_ref[...].astype(bf), w1_ref[...].astype(bf),
                preferred_element_type=jnp.float32) + b1_ref[...]
    h = jnp.maximum(h, 0.0)
    h = jnp.dot(h.astype(bf), w2_ref[...].astype(bf),
                preferred_element_type=jnp.float32) + b2_ref[...]
    h = jnp.maximum(h, 0.0)
    h = jnp.dot(h.astype(bf), w3_ref[...].astype(bf),
                preferred_element_type=jnp.float32) + b3_ref[...]
    h = jnp.maximum(h, 0.0)
    out_ref[...] = jnp.dot(h.astype(bf), w4_ref[...].astype(bf),
                           preferred_element_type=jnp.float32) + b4_ref[...]


def kernel(x, w1, b1, w2, b2, w3, b3, w4, b4):
    B, F = x.shape
    out_features = w4.shape[1]

    TB = min(2048, _round_up(B, 8))
    B_pad = _round_up(B, TB)
    if B_pad != B:
        x = jnp.zeros((B_pad, F), x.dtype).at[:B].set(x)

    grid = (B_pad // TB,)
    flops = 2 * B_pad * (F * w1.shape[1] + w2.size + w3.size + w4.size)
    bytes_accessed = 4 * (B_pad * F + w1.size + w2.size + w3.size + w4.size
                          + B_pad * out_features)

    whole = lambda shape: pl.BlockSpec(shape, lambda i: tuple(0 for _ in shape))

    out = pl.pallas_call(
        _mlp_kernel,
        out_shape=jax.ShapeDtypeStruct((B_pad, out_features), jnp.float32),
        grid=grid,
        in_specs=[
            pl.BlockSpec((TB, F), lambda i: (i, 0)),
            whole(w1.shape), whole(b1.shape),
            whole(w2.shape), whole(b2.shape),
            whole(w3.shape), whole(b3.shape),
            whole(w4.shape), whole(b4.shape),
        ],
        out_specs=pl.BlockSpec((TB, out_features), lambda i: (i, 0)),
        compiler_params=pltpu.CompilerParams(
            dimension_semantics=("parallel",),
            vmem_limit_bytes=64 * 1024 * 1024,
        ),
        cost_estimate=pl.CostEstimate(
            flops=flops, transcendentals=0, bytes_accessed=bytes_accessed),
    )(x, w1, b1, w2, b2, w3, b3, w4, b4)

    return out[:B]


# speedup vs baseline: 1.6204x; 1.6204x over previous
import jax
import jax.numpy as jnp
from jax.experimental import pallas as pl
from jax.experimental.pallas import tpu as pltpu


def _round_up(n, m):
    return (n + m - 1) // m * m


def _mlp_kernel(x_ref, w1t_ref, b1_ref, w2_ref, b2_ref, w3t_ref, b3_ref,
                w4_ref, b4_ref, out_ref):
    bf = jnp.bfloat16
    f32 = jnp.float32
    h = jax.lax.dot_general(
        x_ref[...].astype(bf), w1t_ref[...].astype(bf),
        (((1,), (1,)), ((), ())), preferred_element_type=f32) + b1_ref[...]
    h = jnp.maximum(h, 0.0)
    h = jnp.dot(h.astype(bf), w2_ref[...].astype(bf),
                preferred_element_type=f32) + b2_ref[...]
    h = jnp.maximum(h, 0.0)
    h = jax.lax.dot_general(
        h.astype(bf), w3t_ref[...].astype(bf),
        (((1,), (1,)), ((), ())), preferred_element_type=f32) + b3_ref[...]
    h = jnp.maximum(h, 0.0)
    out = jax.lax.dot_general(
        w4_ref[...].astype(bf), h.astype(bf),
        (((0,), (1,)), ((), ())), preferred_element_type=f32)
    out_ref[...] = out + jnp.transpose(b4_ref[...])


def kernel(x, w1, b1, w2, b2, w3, b3, w4, b4):
    B, F = x.shape
    out_features = w4.shape[1]

    TB = min(2048, _round_up(B, 8))
    B_pad = _round_up(B, TB)
    if B_pad != B:
        x = jnp.zeros((B_pad, F), x.dtype).at[:B].set(x)

    w1t = jnp.transpose(w1)
    w3t = jnp.transpose(w3)

    grid = (B_pad // TB,)
    flops = 2 * B_pad * (F * w1.shape[1] + w2.size + w3.size + w4.size)
    bytes_accessed = 4 * (B_pad * F + w1.size + w2.size + w3.size + w4.size
                          + B_pad * out_features)

    whole = lambda shape: pl.BlockSpec(shape, lambda i: tuple(0 for _ in shape))

    out_t = pl.pallas_call(
        _mlp_kernel,
        out_shape=jax.ShapeDtypeStruct((out_features, B_pad), jnp.float32),
        grid=grid,
        in_specs=[
            pl.BlockSpec((TB, F), lambda i: (i, 0)),
            whole(w1t.shape), whole(b1.shape),
            whole(w2.shape), whole(b2.shape),
            whole(w3t.shape), whole(b3.shape),
            whole(w4.shape), whole(b4.shape),
        ],
        out_specs=pl.BlockSpec((out_features, TB), lambda i: (0, i)),
        compiler_params=pltpu.CompilerParams(
            dimension_semantics=("parallel",),
            vmem_limit_bytes=64 * 1024 * 1024,
        ),
        cost_estimate=pl.CostEstimate(
            flops=flops, transcendentals=0, bytes_accessed=bytes_accessed),
    )(x, w1t, b1, w2, b2, w3t, b3, w4, b4)

    return out_t[:, :B].T


# speedup vs baseline: 1.6838x; 1.0391x over previous
import jax
import jax.numpy as jnp
from jax.experimental import pallas as pl
from jax.experimental.pallas import tpu as pltpu


def _round_up(n, m):
    return (n + m - 1) // m * m


def _mlp_kernel(x_ref, w1t_ref, b1_ref, w2_ref, b2_ref, w3t_ref, b3_ref,
                w4_ref, b4_ref, out_ref):
    bf = jnp.bfloat16
    f32 = jnp.float32
    h = jax.lax.dot_general(
        x_ref[...].astype(bf), w1t_ref[...].astype(bf),
        (((1,), (1,)), ((), ())), preferred_element_type=f32) + b1_ref[...]
    h = jnp.maximum(h, 0.0)
    h = jnp.dot(h.astype(bf), w2_ref[...].astype(bf),
                preferred_element_type=f32) + b2_ref[...]
    h = jnp.maximum(h, 0.0)
    h = jax.lax.dot_general(
        h.astype(bf), w3t_ref[...].astype(bf),
        (((1,), (1,)), ((), ())), preferred_element_type=f32) + b3_ref[...]
    h = jnp.maximum(h, 0.0)
    out = jax.lax.dot_general(
        w4_ref[...].astype(bf), h.astype(bf),
        (((0,), (1,)), ((), ())), preferred_element_type=f32)
    out_ref[...] = out + jnp.transpose(b4_ref[...])


def kernel(x, w1, b1, w2, b2, w3, b3, w4, b4):
    B, F = x.shape
    out_features = w4.shape[1]

    TB = min(4096, _round_up(B, 8))
    B_pad = _round_up(B, TB)
    if B_pad != B:
        x = jnp.zeros((B_pad, F), x.dtype).at[:B].set(x)

    w1t = jnp.transpose(w1)
    w3t = jnp.transpose(w3)

    grid = (B_pad // TB,)
    flops = 2 * B_pad * (F * w1.shape[1] + w2.size + w3.size + w4.size)
    bytes_accessed = 4 * (B_pad * F + w1.size + w2.size + w3.size + w4.size
                          + B_pad * out_features)

    whole = lambda shape: pl.BlockSpec(shape, lambda i: tuple(0 for _ in shape))

    out_t = pl.pallas_call(
        _mlp_kernel,
        out_shape=jax.ShapeDtypeStruct((out_features, B_pad), jnp.float32),
        grid=grid,
        in_specs=[
            pl.BlockSpec((TB, F), lambda i: (i, 0)),
            whole(w1t.shape), whole(b1.shape),
            whole(w2.shape), whole(b2.shape),
            whole(w3t.shape), whole(b3.shape),
            whole(w4.shape), whole(b4.shape),
        ],
        out_specs=pl.BlockSpec((out_features, TB), lambda i: (0, i)),
        compiler_params=pltpu.CompilerParams(
            dimension_semantics=("parallel",),
            vmem_limit_bytes=64 * 1024 * 1024,
        ),
        cost_estimate=pl.CostEstimate(
            flops=flops, transcendentals=0, bytes_accessed=bytes_accessed),
    )(x, w1t, b1, w2, b2, w3t, b3, w4, b4)

    return out_t[:, :B].T
